# baseline (speedup 1.0000x reference)
import functools

import jax
import jax.numpy as jnp
from jax import lax
from jax.experimental import pallas as pl
from jax.experimental.pallas import tpu as pltpu
from jax.experimental.pallas import tpu_sc as plsc

EMBED = 64
WIDE = 128
SCALE = 8.0
NC = 2
NS = 16
NW = NC * NS
CHUNK = 128
LANES = 16


def _transpose_block(src_v, dst_v, n_rows, scale):
    lane = lax.iota(jnp.int32, LANES)
    n_cols = src_v.shape[-1]

    @plsc.parallel_loop(0, n_rows, 1, unroll=4)
    def rows(r):
        rcol = jnp.broadcast_to(r, (LANES,)).astype(jnp.int32)
        for d in range(n_cols // LANES):
            vec = src_v[r, pl.ds(d * LANES, LANES)]
            plsc.store_scatter(dst_v, [lane + d * LANES, rcol], vec * scale)


@jax.jit
def _table_rowmajor(table_v):
    vocab = table_v.shape[0]
    n_blocks = vocab // EMBED
    stride = n_blocks
    n_iter = (n_blocks + NW - 1) // NW
    if n_iter % 2:
        n_iter += 1
    mesh = plsc.VectorSubcoreMesh(core_axis_name="c", subcore_axis_name="s")

    @functools.partial(
        pl.kernel,
        mesh=mesh,
        out_type=jax.ShapeDtypeStruct((vocab, EMBED), jnp.float32),
        scratch_types=[
            pltpu.VMEM((2, EMBED), jnp.int32),
            pltpu.VMEM((2, EMBED, EMBED), jnp.float32),
            pltpu.VMEM((2, EMBED, EMBED + 1), jnp.float32),
            pltpu.SemaphoreType.DMA,
            pltpu.SemaphoreType.DMA,
            pltpu.SemaphoreType.DMA,
            pltpu.SemaphoreType.DMA,
        ],
        compiler_params=pltpu.CompilerParams(
            use_tc_tiling_on_sc=False, needs_layout_passes=False
        ),
    )
    def body(tv_hbm, rm_hbm, idx_v, stage_v, trans_v, g0, g1, w0, w1):
        gsem = (g0, g1)
        wsem = (w0, w1)
        lane = lax.iota(jnp.int32, LANES)
        wid = lax.axis_index("s") * NC + lax.axis_index("c")

        def fill_idx(buf, blk):
            for k in range(EMBED // LANES):
                idx_v[buf, pl.ds(k * LANES, LANES)] = (
                    (lane + k * LANES) * stride + blk
                )

        fill_idx(0, wid)
        pltpu.async_copy(tv_hbm.at[idx_v.at[0]], stage_v.at[0], gsem[0])

        @pl.loop(0, n_iter, step=2)
        def outer(i0):
            for b in range(2):
                i = i0 + b
                other = 1 - b
                blk = wid + i * NW
                nxt = blk + NW

                @pl.when(nxt < n_blocks)
                def _():
                    fill_idx(other, nxt)
                    pltpu.async_copy(
                        tv_hbm.at[idx_v.at[other]], stage_v.at[other],
                        gsem[other],
                    )

                @pl.when(blk < n_blocks)
                def _():
                    pltpu.make_async_copy(
                        tv_hbm.at[pl.ds(0, EMBED)], stage_v.at[b], gsem[b]
                    ).wait()

                    @pl.when(i >= 2)
                    def _():
                        pltpu.make_async_copy(
                            trans_v.at[b, :, pl.ds(0, EMBED)],
                            rm_hbm.at[pl.ds(0, EMBED)],
                            wsem[b],
                        ).wait()

                    _transpose_block(stage_v.at[b], trans_v.at[b], EMBED, 1.0)
                    pltpu.async_copy(
                        trans_v.at[b, :, pl.ds(0, EMBED)],
                        rm_hbm.at[pl.ds(blk * EMBED, EMBED)],
                        wsem[b],
                    )

        for b in range(2):
            pltpu.make_async_copy(
                trans_v.at[b, :, pl.ds(0, EMBED)],
                rm_hbm.at[pl.ds(0, EMBED)],
                wsem[b],
            ).wait()

    return body(table_v)


@functools.partial(jax.jit, static_argnames=("seq", "batch"))
def _emb_lookup(tok_t, table_rm, seq, batch):
    mesh = plsc.VectorSubcoreMesh(core_axis_name="c", subcore_axis_name="s")

    @functools.partial(
        pl.kernel,
        mesh=mesh,
        out_type=jax.ShapeDtypeStruct((seq, EMBED, batch), jnp.float32),
        scratch_types=[
            pltpu.VMEM((seq, CHUNK), jnp.int32),
            pltpu.VMEM((2, CHUNK, EMBED), jnp.float32),
            pltpu.VMEM((2, EMBED, CHUNK + 1), jnp.float32),
            pltpu.SemaphoreType.DMA,
            pltpu.SemaphoreType.DMA,
            pltpu.SemaphoreType.DMA,
            pltpu.SemaphoreType.DMA,
        ],
        compiler_params=pltpu.CompilerParams(
            use_tc_tiling_on_sc=False, needs_layout_passes=False
        ),
    )
    def body(tok_hbm, table_hbm, out_hbm, idx_v, wide_v, trans_v, g0, g1, w0, w1):
        gsem = (g0, g1)
        wsem = (w0, w1)
        wid = lax.axis_index("s") * NC + lax.axis_index("c")
        col0 = wid * CHUNK
        pltpu.sync_copy(tok_hbm.at[:, pl.ds(col0, CHUNK)], idx_v)

        pltpu.async_copy(table_hbm.at[idx_v.at[0]], wide_v.at[0], gsem[0])

        @pl.loop(0, seq, step=2)
        def outer(j0):
            for b in range(2):
                j = j0 + b
                other = 1 - b

                @pl.when(j + 1 < seq)
                def _():
                    pltpu.async_copy(
                        table_hbm.at[idx_v.at[j + 1]], wide_v.at[other],
                        gsem[other],
                    )

                pltpu.make_async_copy(
                    table_hbm.at[pl.ds(0, CHUNK)], wide_v.at[b], gsem[b]
                ).wait()

                @pl.when(j >= 2)
                def _():
                    pltpu.make_async_copy(
                        trans_v.at[b, :, pl.ds(0, CHUNK)],
                        out_hbm.at[0, :, pl.ds(0, CHUNK)],
                        wsem[b],
                    ).wait()

                _transpose_block(wide_v.at[b], trans_v.at[b], CHUNK, SCALE)

                pltpu.async_copy(
                    trans_v.at[b, :, pl.ds(0, CHUNK)],
                    out_hbm.at[j, :, pl.ds(col0, CHUNK)],
                    wsem[b],
                )

        for b in range(2):
            pltpu.make_async_copy(
                trans_v.at[b, :, pl.ds(0, CHUNK)],
                out_hbm.at[0, :, pl.ds(0, CHUNK)],
                wsem[b],
            ).wait()

    return body(tok_t, table_rm)


def kernel(tokens, table):
    b, s = tokens.shape
    tok_t = tokens.T.astype(jnp.int32)
    table_view = lax.reshape(table, (table.shape[0], EMBED), dimensions=(1, 0))
    table_rm = _table_rowmajor(table_view)
    out = _emb_lookup(tok_t, table_rm, s, b)
    return out.transpose(2, 0, 1)


# speedup vs baseline: 1.0141x; 1.0141x over previous
import functools

import jax
import jax.numpy as jnp
from jax import lax
from jax.experimental import pallas as pl
from jax.experimental.pallas import tpu as pltpu
from jax.experimental.pallas import tpu_sc as plsc

EMBED = 64
WIDE = 128
SCALE = 8.0
NC = 2
NS = 16
NW = NC * NS
CHUNK = 128
LANES = 16


def _transpose_block(src_v, dst_v, n_rows, scale):
    lane = lax.iota(jnp.int32, LANES)
    n_cols = src_v.shape[-1]

    @plsc.parallel_loop(0, n_rows, 1, unroll=4)
    def rows(r):
        rcol = jnp.broadcast_to(r, (LANES,)).astype(jnp.int32)
        for d in range(n_cols // LANES):
            vec = src_v[r, pl.ds(d * LANES, LANES)]
            plsc.store_scatter(dst_v, [lane + d * LANES, rcol], vec * scale)


@jax.jit
def _table_rowmajor(table_t):
    vocab = table_t.shape[1]
    cw = 256
    n_full = vocab // cw
    tail = vocab - n_full * cw
    n_iter = (n_full + NW - 1) // NW
    if n_iter % 2:
        n_iter += 1
    mesh = plsc.VectorSubcoreMesh(core_axis_name="c", subcore_axis_name="s")

    @functools.partial(
        pl.kernel,
        mesh=mesh,
        out_type=jax.ShapeDtypeStruct((vocab, EMBED), jnp.float32),
        scratch_types=[
            pltpu.VMEM((2, EMBED, cw), jnp.float32),
            pltpu.VMEM((2, cw, EMBED + 1), jnp.float32),
            pltpu.VMEM((EMBED, 64), jnp.float32),
            pltpu.VMEM((64, EMBED + 1), jnp.float32),
            pltpu.SemaphoreType.DMA,
            pltpu.SemaphoreType.DMA,
            pltpu.SemaphoreType.DMA,
            pltpu.SemaphoreType.DMA,
        ],
        compiler_params=pltpu.CompilerParams(
            use_tc_tiling_on_sc=False, needs_layout_passes=False
        ),
    )
    def body(tt_hbm, rm_hbm, stage_v, trans_v, tstage_v, ttrans_v,
             g0, g1, w0, w1):
        gsem = (g0, g1)
        wsem = (w0, w1)
        wid = lax.axis_index("s") * NC + lax.axis_index("c")

        def stage_chunk(buf, blk):
            for e in range(EMBED):
                pltpu.async_copy(
                    tt_hbm.at[e, pl.ds(blk * cw, cw)],
                    stage_v.at[buf, e],
                    gsem[buf],
                )

        stage_chunk(0, wid)

        @pl.loop(0, n_iter, step=2)
        def outer(i0):
            for b in range(2):
                i = i0 + b
                other = 1 - b
                blk = wid + i * NW
                nxt = blk + NW

                @pl.when(nxt < n_full)
                def _():
                    stage_chunk(other, nxt)

                @pl.when(blk < n_full)
                def _():
                    pltpu.make_async_copy(
                        tt_hbm.at[:, pl.ds(0, cw)], stage_v.at[b], gsem[b]
                    ).wait()

                    @pl.when(i >= 2)
                    def _():
                        pltpu.make_async_copy(
                            trans_v.at[b, :, pl.ds(0, EMBED)],
                            rm_hbm.at[pl.ds(0, cw)],
                            wsem[b],
                        ).wait()

                    _transpose_block(stage_v.at[b], trans_v.at[b], EMBED, 1.0)
                    pltpu.async_copy(
                        trans_v.at[b, :, pl.ds(0, EMBED)],
                        rm_hbm.at[pl.ds(blk * cw, cw)],
                        wsem[b],
                    )

        for b in range(2):
            pltpu.make_async_copy(
                trans_v.at[b, :, pl.ds(0, EMBED)],
                rm_hbm.at[pl.ds(0, cw)],
                wsem[b],
            ).wait()

        if tail:
            assert tail == 64
            @pl.when(wid == 0)
            def _():
                for e in range(EMBED):
                    pltpu.async_copy(
                        tt_hbm.at[e, pl.ds(n_full * cw, tail)],
                        tstage_v.at[e],
                        g0,
                    )
                pltpu.make_async_copy(
                    tt_hbm.at[:, pl.ds(0, tail)], tstage_v, g0
                ).wait()
                _transpose_block(tstage_v, ttrans_v, EMBED, 1.0)
                pltpu.sync_copy(
                    ttrans_v.at[:, pl.ds(0, EMBED)],
                    rm_hbm.at[pl.ds(n_full * cw, tail)],
                )

    return body(table_t)


@functools.partial(jax.jit, static_argnames=("seq", "batch"))
def _emb_lookup(tok_t, table_rm, seq, batch):
    mesh = plsc.VectorSubcoreMesh(core_axis_name="c", subcore_axis_name="s")

    @functools.partial(
        pl.kernel,
        mesh=mesh,
        out_type=jax.ShapeDtypeStruct((seq, EMBED, batch), jnp.float32),
        scratch_types=[
            pltpu.VMEM((seq, CHUNK), jnp.int32),
            pltpu.VMEM((2, CHUNK, EMBED), jnp.float32),
            pltpu.VMEM((2, EMBED, CHUNK + 1), jnp.float32),
            pltpu.SemaphoreType.DMA,
            pltpu.SemaphoreType.DMA,
            pltpu.SemaphoreType.DMA,
            pltpu.SemaphoreType.DMA,
        ],
        compiler_params=pltpu.CompilerParams(
            use_tc_tiling_on_sc=False, needs_layout_passes=False
        ),
    )
    def body(tok_hbm, table_hbm, out_hbm, idx_v, wide_v, trans_v, g0, g1, w0, w1):
        gsem = (g0, g1)
        wsem = (w0, w1)
        wid = lax.axis_index("s") * NC + lax.axis_index("c")
        col0 = wid * CHUNK
        pltpu.sync_copy(tok_hbm.at[:, pl.ds(col0, CHUNK)], idx_v)

        pltpu.async_copy(table_hbm.at[idx_v.at[0]], wide_v.at[0], gsem[0])

        @pl.loop(0, seq, step=2)
        def outer(j0):
            for b in range(2):
                j = j0 + b
                other = 1 - b

                @pl.when(j + 1 < seq)
                def _():
                    pltpu.async_copy(
                        table_hbm.at[idx_v.at[j + 1]], wide_v.at[other],
                        gsem[other],
                    )

                pltpu.make_async_copy(
                    table_hbm.at[pl.ds(0, CHUNK)], wide_v.at[b], gsem[b]
                ).wait()

                @pl.when(j >= 2)
                def _():
                    pltpu.make_async_copy(
                        trans_v.at[b, :, pl.ds(0, CHUNK)],
                        out_hbm.at[0, :, pl.ds(0, CHUNK)],
                        wsem[b],
                    ).wait()

                _transpose_block(wide_v.at[b], trans_v.at[b], CHUNK, SCALE)

                pltpu.async_copy(
                    trans_v.at[b, :, pl.ds(0, CHUNK)],
                    out_hbm.at[j, :, pl.ds(col0, CHUNK)],
                    wsem[b],
                )

        for b in range(2):
            pltpu.make_async_copy(
                trans_v.at[b, :, pl.ds(0, CHUNK)],
                out_hbm.at[0, :, pl.ds(0, CHUNK)],
                wsem[b],
            ).wait()

    return body(tok_t, table_rm)


def kernel(tokens, table):
    b, s = tokens.shape
    tok_t = tokens.T.astype(jnp.int32)
    table_rm = _table_rowmajor(table.T)
    out = _emb_lookup(tok_t, table_rm, s, b)
    return out.transpose(2, 0, 1)


# speedup vs baseline: 5.7503x; 5.6704x over previous
import functools

import jax
import jax.numpy as jnp
from jax import lax
from jax.experimental import pallas as pl
from jax.experimental.pallas import tpu as pltpu
from jax.experimental.pallas import tpu_sc as plsc

EMBED = 64
WIDE = 128
SCALE = 8.0
NC = 2
NS = 16
NW = NC * NS
CHUNK = 128
LANES = 16


def _transpose_block(src_v, dst_v, n_rows, scale, n_cols=None):
    lane = lax.iota(jnp.int32, LANES)
    if n_cols is None:
        n_cols = src_v.shape[-1]

    @plsc.parallel_loop(0, n_rows, 1, unroll=4)
    def rows(r):
        rcol = jnp.broadcast_to(r, (LANES,)).astype(jnp.int32)
        for d in range(n_cols // LANES):
            vec = src_v[r, pl.ds(d * LANES, LANES)]
            plsc.store_scatter(dst_v, [lane + d * LANES, rcol], vec * scale)


@jax.jit
def _table_rowmajor(table_t):
    vocab = table_t.shape[1]
    cw = 256
    n_full = vocab // cw
    tail = vocab - n_full * cw
    n_iter = (n_full + NW - 1) // NW
    if n_iter % 2:
        n_iter += 1
    mesh = plsc.VectorSubcoreMesh(core_axis_name="c", subcore_axis_name="s")

    @functools.partial(
        pl.kernel,
        mesh=mesh,
        out_type=jax.ShapeDtypeStruct((vocab, EMBED), jnp.float32),
        scratch_types=[
            pltpu.VMEM((2, EMBED, cw), jnp.float32),
            pltpu.VMEM((2, cw, EMBED + 1), jnp.float32),
            pltpu.VMEM((EMBED, 64), jnp.float32),
            pltpu.VMEM((64, EMBED + 1), jnp.float32),
            pltpu.SemaphoreType.DMA,
            pltpu.SemaphoreType.DMA,
            pltpu.SemaphoreType.DMA,
            pltpu.SemaphoreType.DMA,
        ],
        compiler_params=pltpu.CompilerParams(
            use_tc_tiling_on_sc=False, needs_layout_passes=False
        ),
    )
    def body(tt_hbm, rm_hbm, stage_v, trans_v, tstage_v, ttrans_v,
             g0, g1, w0, w1):
        gsem = (g0, g1)
        wsem = (w0, w1)
        wid = lax.axis_index("s") * NC + lax.axis_index("c")

        def stage_chunk(buf, blk):
            for e in range(EMBED):
                pltpu.async_copy(
                    tt_hbm.at[e, pl.ds(blk * cw, cw)],
                    stage_v.at[buf, e],
                    gsem[buf],
                )

        stage_chunk(0, wid)

        @pl.loop(0, n_iter, step=2)
        def outer(i0):
            for b in range(2):
                i = i0 + b
                other = 1 - b
                blk = wid + i * NW
                nxt = blk + NW

                @pl.when(nxt < n_full)
                def _():
                    stage_chunk(other, nxt)

                @pl.when(blk < n_full)
                def _():
                    pltpu.make_async_copy(
                        tt_hbm.at[:, pl.ds(0, cw)], stage_v.at[b], gsem[b]
                    ).wait()

                    @pl.when(i >= 2)
                    def _():
                        pltpu.make_async_copy(
                            trans_v.at[b, :, pl.ds(0, EMBED)],
                            rm_hbm.at[pl.ds(0, cw)],
                            wsem[b],
                        ).wait()

                    _transpose_block(stage_v.at[b], trans_v.at[b], EMBED, 1.0)
                    pltpu.async_copy(
                        trans_v.at[b, :, pl.ds(0, EMBED)],
                        rm_hbm.at[pl.ds(blk * cw, cw)],
                        wsem[b],
                    )

        for b in range(2):
            pltpu.make_async_copy(
                trans_v.at[b, :, pl.ds(0, EMBED)],
                rm_hbm.at[pl.ds(0, cw)],
                wsem[b],
            ).wait()

        if tail:
            assert tail == 64
            @pl.when(wid == 0)
            def _():
                for e in range(EMBED):
                    pltpu.async_copy(
                        tt_hbm.at[e, pl.ds(n_full * cw, tail)],
                        tstage_v.at[e],
                        g0,
                    )
                pltpu.make_async_copy(
                    tt_hbm.at[:, pl.ds(0, tail)], tstage_v, g0
                ).wait()
                _transpose_block(tstage_v, ttrans_v, EMBED, 1.0)
                pltpu.sync_copy(
                    ttrans_v.at[:, pl.ds(0, EMBED)],
                    rm_hbm.at[pl.ds(n_full * cw, tail)],
                )

    return body(table_t)


@functools.partial(jax.jit, static_argnames=("seq", "batch"))
def _emb_lookup(tok_t, table_rm, seq, batch):
    mesh = plsc.VectorSubcoreMesh(core_axis_name="c", subcore_axis_name="s")

    @functools.partial(
        pl.kernel,
        mesh=mesh,
        out_type=jax.ShapeDtypeStruct((seq, EMBED, batch), jnp.float32),
        scratch_types=[
            pltpu.VMEM((seq, CHUNK), jnp.int32),
            pltpu.VMEM((2, CHUNK, WIDE), jnp.float32),
            pltpu.VMEM((2, EMBED, CHUNK + 1), jnp.float32),
            pltpu.SemaphoreType.DMA,
            pltpu.SemaphoreType.DMA,
            pltpu.SemaphoreType.DMA,
            pltpu.SemaphoreType.DMA,
        ],
        compiler_params=pltpu.CompilerParams(
            use_tc_tiling_on_sc=False, needs_layout_passes=False
        ),
    )
    def body(tok_hbm, table_hbm, out_hbm, idx_v, wide_v, trans_v, g0, g1, w0, w1):
        gsem = (g0, g1)
        wsem = (w0, w1)
        wid = lax.axis_index("s") * NC + lax.axis_index("c")
        col0 = wid * CHUNK
        pltpu.sync_copy(tok_hbm.at[:, pl.ds(col0, CHUNK)], idx_v)

        pltpu.async_copy(table_hbm.at[idx_v.at[0]], wide_v.at[0], gsem[0])

        @pl.loop(0, seq, step=2)
        def outer(j0):
            for b in range(2):
                j = j0 + b
                other = 1 - b

                @pl.when(j + 1 < seq)
                def _():
                    pltpu.async_copy(
                        table_hbm.at[idx_v.at[j + 1]], wide_v.at[other],
                        gsem[other],
                    )

                pltpu.make_async_copy(
                    table_hbm.at[pl.ds(0, CHUNK)], wide_v.at[b], gsem[b]
                ).wait()

                @pl.when(j >= 2)
                def _():
                    pltpu.make_async_copy(
                        trans_v.at[b, :, pl.ds(0, CHUNK)],
                        out_hbm.at[0, :, pl.ds(0, CHUNK)],
                        wsem[b],
                    ).wait()

                _transpose_block(
                    wide_v.at[b], trans_v.at[b], CHUNK, SCALE,
                    n_cols=EMBED,
                )

                pltpu.async_copy(
                    trans_v.at[b, :, pl.ds(0, CHUNK)],
                    out_hbm.at[j, :, pl.ds(col0, CHUNK)],
                    wsem[b],
                )

        for b in range(2):
            pltpu.make_async_copy(
                trans_v.at[b, :, pl.ds(0, CHUNK)],
                out_hbm.at[0, :, pl.ds(0, CHUNK)],
                wsem[b],
            ).wait()

    return body(tok_t, table_rm)


def kernel(tokens, table):
    b, s = tokens.shape
    tok_t = tokens.T.astype(jnp.int32)
    table_wide = jnp.pad(table, ((0, 0), (0, WIDE - EMBED)))
    out = _emb_lookup(tok_t, table_wide, s, b)
    return out.transpose(2, 0, 1)
